# baseline (device time: 66951 ns/iter reference)
import jax
import jax.numpy as jnp
from jax import lax
from jax.experimental import pallas as pl
from jax.experimental.pallas import tpu as pltpu

N_DEV = 8
SUB = 2


def kernel(A, B):
    M, K = A.shape
    K2, N = B.shape
    assert K == K2
    CHUNK = M // N_DEV
    HALF = N // 2
    ROWS = CHUNK // SUB

    def body(a_ref, b_ref, out_ref, b_bf,
             send_r, send_l, recv_r, recv_l,
             send_sems_r, send_sems_l, recv_sems_r, recv_sems_l):
        my = lax.axis_index("i")
        left = lax.rem(my + N_DEV - 1, N_DEV)
        right = lax.rem(my + 1, N_DEV)

        barrier_sem = pltpu.get_barrier_semaphore()
        for nbr in (left, right):
            pl.semaphore_signal(
                barrier_sem, inc=1,
                device_id=(nbr,), device_id_type=pl.DeviceIdType.MESH,
            )
        pl.semaphore_wait(barrier_sem, 2)

        def pchunk(c, lo, hi):
            a_c = a_ref[pl.ds(c * CHUNK, CHUNK), :].astype(jnp.bfloat16)
            return lax.dot_general(
                a_c, b_bf[:, lo:hi], (((1,), (0,)), ((), ())),
                preferred_element_type=jnp.float32,
            )

        def mk_rdma(direction, s, u):
            if direction == "r":
                return pltpu.make_async_remote_copy(
                    src_ref=send_r.at[s % 2, u], dst_ref=recv_r.at[s, u],
                    send_sem=send_sems_r.at[s % 2, u],
                    recv_sem=recv_sems_r.at[s, u],
                    device_id=(right,), device_id_type=pl.DeviceIdType.MESH,
                )
            return pltpu.make_async_remote_copy(
                src_ref=send_l.at[s % 2, u], dst_ref=recv_l.at[s, u],
                send_sem=send_sems_l.at[s % 2, u],
                recv_sem=recv_sems_l.at[s, u],
                device_id=(left,), device_id_type=pl.DeviceIdType.MESH,
            )

        def stage(direction, s, acc):
            buf = send_r if direction == "r" else send_l
            for u in range(SUB):
                buf[s % 2, u] = acc[u * ROWS:(u + 1) * ROWS, :].astype(
                    jnp.bfloat16)
                mk_rdma(direction, s, u).start()


        b_bf[:, :HALF] = b_ref[:, :HALF].astype(jnp.bfloat16)
        stage("r", 0, pchunk(left, 0, HALF))
        b_bf[:, HALF:] = b_ref[:, HALF:].astype(jnp.bfloat16)
        stage("l", 0, pchunk(right, HALF, N))

        for s in range(N_DEV - 1):
            c_r = lax.rem(my + (2 * N_DEV - 2 - s), N_DEV)
            c_l = lax.rem(my + 2 + s, N_DEV)
            p_r = pchunk(c_r, 0, HALF)
            p_l = pchunk(c_l, HALF, N)
            if s >= 1:
                for u in range(SUB):
                    mk_rdma("r", s - 1, u).wait_send()
                    mk_rdma("l", s - 1, u).wait_send()
            if s < N_DEV - 2:
                for u in range(SUB):
                    rows = slice(u * ROWS, (u + 1) * ROWS)
                    mk_rdma("r", s, u).wait_recv()
                    acc_u = recv_r[s, u].astype(jnp.float32) + p_r[rows, :]
                    send_r[(s + 1) % 2, u] = acc_u.astype(jnp.bfloat16)
                    mk_rdma("r", s + 1, u).start()
                    mk_rdma("l", s, u).wait_recv()
                    acc_u = recv_l[s, u].astype(jnp.float32) + p_l[rows, :]
                    send_l[(s + 1) % 2, u] = acc_u.astype(jnp.bfloat16)
                    mk_rdma("l", s + 1, u).start()
            else:
                for u in range(SUB):
                    rows = slice(u * ROWS, (u + 1) * ROWS)
                    mk_rdma("r", s, u).wait_recv()
                    out_ref[rows, :HALF] = (
                        recv_r[s, u].astype(jnp.float32) + p_r[rows, :])
                    mk_rdma("l", s, u).wait_recv()
                    out_ref[rows, HALF:] = (
                        recv_l[s, u].astype(jnp.float32) + p_l[rows, :])

        for u in range(SUB):
            mk_rdma("r", N_DEV - 2, u).wait_send()
            mk_rdma("l", N_DEV - 2, u).wait_send()

    return pl.pallas_call(
        body,
        out_shape=jax.ShapeDtypeStruct((CHUNK, N), jnp.float32),
        in_specs=[
            pl.BlockSpec(memory_space=pltpu.VMEM),
            pl.BlockSpec(memory_space=pltpu.VMEM),
        ],
        out_specs=pl.BlockSpec(memory_space=pltpu.VMEM),
        scratch_shapes=[
            pltpu.VMEM((K, N), jnp.bfloat16),
            pltpu.VMEM((2, SUB, ROWS, HALF), jnp.bfloat16),
            pltpu.VMEM((2, SUB, ROWS, HALF), jnp.bfloat16),
            pltpu.VMEM((N_DEV - 1, SUB, ROWS, HALF), jnp.bfloat16),
            pltpu.VMEM((N_DEV - 1, SUB, ROWS, HALF), jnp.bfloat16),
            pltpu.SemaphoreType.DMA((2, SUB)),
            pltpu.SemaphoreType.DMA((2, SUB)),
            pltpu.SemaphoreType.DMA((N_DEV - 1, SUB)),
            pltpu.SemaphoreType.DMA((N_DEV - 1, SUB)),
        ],
        compiler_params=pltpu.CompilerParams(collective_id=0),
    )(A, B)


# device time: 54928 ns/iter; 1.2189x vs baseline; 1.2189x over previous
import jax
import jax.numpy as jnp
from jax import lax
from jax.experimental import pallas as pl
from jax.experimental.pallas import tpu as pltpu

N_DEV = 8
SUB = 2


def kernel(A, B):
    M, K = A.shape
    K2, N = B.shape
    assert K == K2
    CHUNK = M // N_DEV
    HALF = N // 2
    ROWS = CHUNK // SUB

    def body(a_ref, b_ref, out_ref, b_bf,
             send_r, send_l, recv_r, recv_l,
             send_sems_r, send_sems_l, recv_sems_r, recv_sems_l):
        my = lax.axis_index("i")
        left = lax.rem(my + N_DEV - 1, N_DEV)
        right = lax.rem(my + 1, N_DEV)

        barrier_sem = pltpu.get_barrier_semaphore()
        for nbr in (left, right):
            pl.semaphore_signal(
                barrier_sem, inc=1,
                device_id=(nbr,), device_id_type=pl.DeviceIdType.MESH,
            )
        pl.semaphore_wait(barrier_sem, 2)

        def pchunk(c, lo, hi, out_dtype=jnp.bfloat16):
            a_c = a_ref[pl.ds(c * CHUNK, CHUNK), :].astype(jnp.bfloat16)
            p = lax.dot_general(
                a_c, b_bf[:, lo:hi], (((1,), (0,)), ((), ())),
                preferred_element_type=jnp.float32,
            )
            return p.astype(out_dtype) if out_dtype != jnp.float32 else p

        def mk_rdma(direction, s, u):
            if direction == "r":
                return pltpu.make_async_remote_copy(
                    src_ref=send_r.at[s % 2, u], dst_ref=recv_r.at[s, u],
                    send_sem=send_sems_r.at[s % 2, u],
                    recv_sem=recv_sems_r.at[s, u],
                    device_id=(right,), device_id_type=pl.DeviceIdType.MESH,
                )
            return pltpu.make_async_remote_copy(
                src_ref=send_l.at[s % 2, u], dst_ref=recv_l.at[s, u],
                send_sem=send_sems_l.at[s % 2, u],
                recv_sem=recv_sems_l.at[s, u],
                device_id=(left,), device_id_type=pl.DeviceIdType.MESH,
            )

        def stage(direction, s, acc):
            buf = send_r if direction == "r" else send_l
            for u in range(SUB):
                buf[s % 2, u] = acc[u * ROWS:(u + 1) * ROWS, :]
                mk_rdma(direction, s, u).start()


        b_bf[:, :HALF] = b_ref[:, :HALF].astype(jnp.bfloat16)
        stage("r", 0, pchunk(left, 0, HALF))
        b_bf[:, HALF:] = b_ref[:, HALF:].astype(jnp.bfloat16)
        stage("l", 0, pchunk(right, HALF, N))

        for s in range(N_DEV - 1):
            c_r = lax.rem(my + (2 * N_DEV - 2 - s), N_DEV)
            c_l = lax.rem(my + 2 + s, N_DEV)
            final = s == N_DEV - 2
            p_r = pchunk(c_r, 0, HALF,
                         jnp.float32 if final else jnp.bfloat16)
            p_l = pchunk(c_l, HALF, N,
                         jnp.float32 if final else jnp.bfloat16)
            if s >= 1:
                for u in range(SUB):
                    mk_rdma("r", s - 1, u).wait_send()
                    mk_rdma("l", s - 1, u).wait_send()
            if not final:
                for u in range(SUB):
                    rows = slice(u * ROWS, (u + 1) * ROWS)
                    mk_rdma("r", s, u).wait_recv()
                    send_r[(s + 1) % 2, u] = recv_r[s, u] + p_r[rows, :]
                    mk_rdma("r", s + 1, u).start()
                    mk_rdma("l", s, u).wait_recv()
                    send_l[(s + 1) % 2, u] = recv_l[s, u] + p_l[rows, :]
                    mk_rdma("l", s + 1, u).start()
            else:
                for u in range(SUB):
                    rows = slice(u * ROWS, (u + 1) * ROWS)
                    mk_rdma("r", s, u).wait_recv()
                    out_ref[rows, :HALF] = (
                        recv_r[s, u].astype(jnp.float32) + p_r[rows, :])
                    mk_rdma("l", s, u).wait_recv()
                    out_ref[rows, HALF:] = (
                        recv_l[s, u].astype(jnp.float32) + p_l[rows, :])

        for u in range(SUB):
            mk_rdma("r", N_DEV - 2, u).wait_send()
            mk_rdma("l", N_DEV - 2, u).wait_send()

    return pl.pallas_call(
        body,
        out_shape=jax.ShapeDtypeStruct((CHUNK, N), jnp.float32),
        in_specs=[
            pl.BlockSpec(memory_space=pltpu.VMEM),
            pl.BlockSpec(memory_space=pltpu.VMEM),
        ],
        out_specs=pl.BlockSpec(memory_space=pltpu.VMEM),
        scratch_shapes=[
            pltpu.VMEM((K, N), jnp.bfloat16),
            pltpu.VMEM((2, SUB, ROWS, HALF), jnp.bfloat16),
            pltpu.VMEM((2, SUB, ROWS, HALF), jnp.bfloat16),
            pltpu.VMEM((N_DEV - 1, SUB, ROWS, HALF), jnp.bfloat16),
            pltpu.VMEM((N_DEV - 1, SUB, ROWS, HALF), jnp.bfloat16),
            pltpu.SemaphoreType.DMA((2, SUB)),
            pltpu.SemaphoreType.DMA((2, SUB)),
            pltpu.SemaphoreType.DMA((N_DEV - 1, SUB)),
            pltpu.SemaphoreType.DMA((N_DEV - 1, SUB)),
        ],
        compiler_params=pltpu.CompilerParams(collective_id=0),
    )(A, B)


# device time: 54809 ns/iter; 1.2215x vs baseline; 1.0022x over previous
import jax
import jax.numpy as jnp
from jax import lax
from jax.experimental import pallas as pl
from jax.experimental.pallas import tpu as pltpu

N_DEV = 8
SUB = 2


def kernel(A, B):
    M, K = A.shape
    K2, N = B.shape
    assert K == K2
    CHUNK = M // N_DEV
    HALF = N // 2
    ROWS = CHUNK // SUB

    def body(a_ref, b_ref, out_ref, b_bf,
             send_r, send_l, recv_r, recv_l,
             send_sems_r, send_sems_l, recv_sems_r, recv_sems_l):
        my = lax.axis_index("i")

        def sig(x):
            return jnp.where(x < 4, x, 11 - x)

        r_pos = sig(my)
        left = sig(lax.rem(r_pos + N_DEV - 1, N_DEV))
        right = sig(lax.rem(r_pos + 1, N_DEV))

        barrier_sem = pltpu.get_barrier_semaphore()
        for nbr in (left, right):
            pl.semaphore_signal(
                barrier_sem, inc=1,
                device_id=(nbr,), device_id_type=pl.DeviceIdType.MESH,
            )
        pl.semaphore_wait(barrier_sem, 2)

        def pchunk(c, lo, hi, out_dtype=jnp.bfloat16):
            a_c = a_ref[pl.ds(c * CHUNK, CHUNK), :].astype(jnp.bfloat16)
            p = lax.dot_general(
                a_c, b_bf[:, lo:hi], (((1,), (0,)), ((), ())),
                preferred_element_type=jnp.float32,
            )
            return p.astype(out_dtype) if out_dtype != jnp.float32 else p

        def mk_rdma(direction, s, u):
            if direction == "r":
                return pltpu.make_async_remote_copy(
                    src_ref=send_r.at[s % 2, u], dst_ref=recv_r.at[s, u],
                    send_sem=send_sems_r.at[s % 2, u],
                    recv_sem=recv_sems_r.at[s, u],
                    device_id=(right,), device_id_type=pl.DeviceIdType.MESH,
                )
            return pltpu.make_async_remote_copy(
                src_ref=send_l.at[s % 2, u], dst_ref=recv_l.at[s, u],
                send_sem=send_sems_l.at[s % 2, u],
                recv_sem=recv_sems_l.at[s, u],
                device_id=(left,), device_id_type=pl.DeviceIdType.MESH,
            )

        def stage(direction, s, acc):
            buf = send_r if direction == "r" else send_l
            for u in range(SUB):
                buf[s % 2, u] = acc[u * ROWS:(u + 1) * ROWS, :]
                mk_rdma(direction, s, u).start()


        b_bf[:, :HALF] = b_ref[:, :HALF].astype(jnp.bfloat16)
        stage("r", 0, pchunk(left, 0, HALF))
        b_bf[:, HALF:] = b_ref[:, HALF:].astype(jnp.bfloat16)
        stage("l", 0, pchunk(right, HALF, N))

        for s in range(N_DEV - 1):
            c_r = sig(lax.rem(r_pos + (2 * N_DEV - 2 - s), N_DEV))
            c_l = sig(lax.rem(r_pos + 2 + s, N_DEV))
            final = s == N_DEV - 2
            p_r = pchunk(c_r, 0, HALF,
                         jnp.float32 if final else jnp.bfloat16)
            p_l = pchunk(c_l, HALF, N,
                         jnp.float32 if final else jnp.bfloat16)
            if s >= 1:
                for u in range(SUB):
                    mk_rdma("r", s - 1, u).wait_send()
                    mk_rdma("l", s - 1, u).wait_send()
            if not final:
                for u in range(SUB):
                    rows = slice(u * ROWS, (u + 1) * ROWS)
                    mk_rdma("r", s, u).wait_recv()
                    send_r[(s + 1) % 2, u] = recv_r[s, u] + p_r[rows, :]
                    mk_rdma("r", s + 1, u).start()
                    mk_rdma("l", s, u).wait_recv()
                    send_l[(s + 1) % 2, u] = recv_l[s, u] + p_l[rows, :]
                    mk_rdma("l", s + 1, u).start()
            else:
                for u in range(SUB):
                    rows = slice(u * ROWS, (u + 1) * ROWS)
                    mk_rdma("r", s, u).wait_recv()
                    out_ref[rows, :HALF] = (
                        recv_r[s, u].astype(jnp.float32) + p_r[rows, :])
                    mk_rdma("l", s, u).wait_recv()
                    out_ref[rows, HALF:] = (
                        recv_l[s, u].astype(jnp.float32) + p_l[rows, :])

        for u in range(SUB):
            mk_rdma("r", N_DEV - 2, u).wait_send()
            mk_rdma("l", N_DEV - 2, u).wait_send()

    return pl.pallas_call(
        body,
        out_shape=jax.ShapeDtypeStruct((CHUNK, N), jnp.float32),
        in_specs=[
            pl.BlockSpec(memory_space=pltpu.VMEM),
            pl.BlockSpec(memory_space=pltpu.VMEM),
        ],
        out_specs=pl.BlockSpec(memory_space=pltpu.VMEM),
        scratch_shapes=[
            pltpu.VMEM((K, N), jnp.bfloat16),
            pltpu.VMEM((2, SUB, ROWS, HALF), jnp.bfloat16),
            pltpu.VMEM((2, SUB, ROWS, HALF), jnp.bfloat16),
            pltpu.VMEM((N_DEV - 1, SUB, ROWS, HALF), jnp.bfloat16),
            pltpu.VMEM((N_DEV - 1, SUB, ROWS, HALF), jnp.bfloat16),
            pltpu.SemaphoreType.DMA((2, SUB)),
            pltpu.SemaphoreType.DMA((2, SUB)),
            pltpu.SemaphoreType.DMA((N_DEV - 1, SUB)),
            pltpu.SemaphoreType.DMA((N_DEV - 1, SUB)),
        ],
        compiler_params=pltpu.CompilerParams(collective_id=0),
    )(A, B)


# device time: 47284 ns/iter; 1.4159x vs baseline; 1.1591x over previous
import jax
import jax.numpy as jnp
from jax import lax
from jax.experimental import pallas as pl
from jax.experimental.pallas import tpu as pltpu

N_DEV = 8
LINKS = (4, 2, 1)
FLIPINV = {2: 3, 1: 5, 4: 6}
RELAY = ((2, 1), (1, 4), (4, 2))


def kernel(A, B):
    M, K = A.shape
    K2, N = B.shape
    assert K == K2
    CHUNK = M // N_DEV
    COLS = {4: (0, 768), 2: (768, 1536), 1: (1536, 2048)}

    def body(a_ref, b_ref, out_ref, b_bf, ap_send,
             p0_recv4, p0_recv2, p0_recv1,
             p1_send, p1_recv, p2_send, p2_recv,
             p0_ssem, p0_rsem, p1_ssem, p1_rsem, p2_ssem, p2_rsem):
        my = lax.axis_index("i")
        p0_recv = {4: p0_recv4, 2: p0_recv2, 1: p0_recv1}
        li = {4: 0, 2: 1, 1: 2}

        z0 = lax.div(my, 4)
        q = lax.rem(my, 4)
        y0 = lax.div(q, 2)
        x0 = lax.rem(q + y0, 2)

        def peer(m):
            px = 1 - x0 if m & 4 else x0
            py = 1 - y0 if m & 2 else y0
            pz = 1 - z0 if m & 1 else z0
            return 4 * pz + 2 * py + lax.rem(px + py, 2)

        barrier_sem = pltpu.get_barrier_semaphore()
        for b in LINKS:
            pl.semaphore_signal(
                barrier_sem, inc=1,
                device_id=(peer(b),), device_id_type=pl.DeviceIdType.MESH,
            )
        pl.semaphore_wait(barrier_sem, 3)

        def pchunk(c, out_dtype=jnp.bfloat16):
            a_c = a_ref[pl.ds(c * CHUNK, CHUNK), :].astype(jnp.bfloat16)
            p = lax.dot_general(
                a_c, b_bf[...], (((1,), (0,)), ((), ())),
                preferred_element_type=jnp.float32,
            )
            return p.astype(out_dtype) if out_dtype != jnp.float32 else p

        def send(buf_at, dst_at, ssem_at, rsem_at, b):
            return pltpu.make_async_remote_copy(
                src_ref=buf_at, dst_ref=dst_at,
                send_sem=ssem_at, recv_sem=rsem_at,
                device_id=(peer(b),), device_id_type=pl.DeviceIdType.MESH,
            )

        b_bf[...] = b_ref[...].astype(jnp.bfloat16)

        ap_send[...] = pchunk(peer(7))
        p0 = {}
        for b in LINKS:
            c0, c1 = COLS[b]
            p0[b] = send(ap_send.at[:, pl.ds(c0, c1 - c0)], p0_recv[b],
                         p0_ssem.at[li[b]], p0_rsem.at[li[b]], b)
            p0[b].start()

        p1 = {}
        for l in LINKS:
            p1_send[li[l]] = pchunk(peer(FLIPINV[l]))
        for l in LINKS:
            a = 7 - FLIPINV[l]
            c0, c1 = COLS[a]
            p0[a].wait_recv()
            p1_send[li[l], :, c0:c1] = (
                p1_send[li[l], :, c0:c1] + p0_recv[a][...])
            p1[l] = send(p1_send.at[li[l]], p1_recv.at[li[l]],
                         p1_ssem.at[li[l]], p1_rsem.at[li[l]], l)
            p1[l].start()

        p2 = {}
        for lin, b in RELAY:
            p2_send[li[b]] = pchunk(peer(b))
        for lin, b in RELAY:
            p1_wait = send(p1_send.at[li[lin]], p1_recv.at[li[lin]],
                           p1_ssem.at[li[lin]], p1_rsem.at[li[lin]], lin)
            p1_wait.wait_recv()
            p2_send[li[b]] = p2_send[li[b]] + p1_recv[li[lin]]
            p2[b] = send(p2_send.at[li[b]], p2_recv.at[li[b]],
                         p2_ssem.at[li[b]], p2_rsem.at[li[b]], b)
            p2[b].start()

        acc = pchunk(my, jnp.float32)
        for b in LINKS:
            wait = send(p2_send.at[li[b]], p2_recv.at[li[b]],
                        p2_ssem.at[li[b]], p2_rsem.at[li[b]], b)
            wait.wait_recv()
            acc = acc + p2_recv[li[b]].astype(jnp.float32)
        out_ref[...] = acc

        for b in LINKS:
            p0[b].wait_send()
            p1[b].wait_send()
            p2[b].wait_send()

    return pl.pallas_call(
        body,
        out_shape=jax.ShapeDtypeStruct((CHUNK, N), jnp.float32),
        in_specs=[
            pl.BlockSpec(memory_space=pltpu.VMEM),
            pl.BlockSpec(memory_space=pltpu.VMEM),
        ],
        out_specs=pl.BlockSpec(memory_space=pltpu.VMEM),
        scratch_shapes=[
            pltpu.VMEM((K, N), jnp.bfloat16),
            pltpu.VMEM((CHUNK, N), jnp.bfloat16),
            pltpu.VMEM((CHUNK, 768), jnp.bfloat16),
            pltpu.VMEM((CHUNK, 768), jnp.bfloat16),
            pltpu.VMEM((CHUNK, 512), jnp.bfloat16),
            pltpu.VMEM((3, CHUNK, N), jnp.bfloat16),
            pltpu.VMEM((3, CHUNK, N), jnp.bfloat16),
            pltpu.VMEM((3, CHUNK, N), jnp.bfloat16),
            pltpu.VMEM((3, CHUNK, N), jnp.bfloat16),
            pltpu.SemaphoreType.DMA((3,)),
            pltpu.SemaphoreType.DMA((3,)),
            pltpu.SemaphoreType.DMA((3,)),
            pltpu.SemaphoreType.DMA((3,)),
            pltpu.SemaphoreType.DMA((3,)),
            pltpu.SemaphoreType.DMA((3,)),
        ],
        compiler_params=pltpu.CompilerParams(collective_id=0),
    )(A, B)


# device time: 44501 ns/iter; 1.5045x vs baseline; 1.0625x over previous
import jax
import jax.numpy as jnp
from jax import lax
from jax.experimental import pallas as pl
from jax.experimental.pallas import tpu as pltpu

N_DEV = 8
LINKS = (4, 2, 1)
FLIPINV = {2: 3, 1: 5, 4: 6}
RELAY = ((2, 1), (1, 4), (4, 2))
COLS = {4: (0, 768), 2: (768, 1536), 1: (1536, 2048)}
PIECES = {
    4: ((0, 1536), (1536, 2048)),
    2: ((768, 2048), (0, 768)),
    1: ((0, 768), (768, 2048)),
}


def kernel(A, B):
    M, K = A.shape
    K2, N = B.shape
    assert K == K2
    CHUNK = M // N_DEV

    def body(a_ref, b_ref, out_ref, b_bf, ap_send,
             p0_recv4, p0_recv2, p0_recv1,
             p1_send, p1_recv, p2_send, p2_recv,
             p0_ssem, p0_rsem, p1_ssem, p1_rsem, p2_ssem, p2_rsem):
        my = lax.axis_index("i")
        p0_recv = {4: p0_recv4, 2: p0_recv2, 1: p0_recv1}
        li = {4: 0, 2: 1, 1: 2}

        z0 = lax.div(my, 4)
        q = lax.rem(my, 4)
        y0 = lax.div(q, 2)
        x0 = lax.rem(q + y0, 2)

        def peer(m):
            px = 1 - x0 if m & 4 else x0
            py = 1 - y0 if m & 2 else y0
            pz = 1 - z0 if m & 1 else z0
            return 4 * pz + 2 * py + lax.rem(px + py, 2)

        barrier_sem = pltpu.get_barrier_semaphore()
        for b in LINKS:
            pl.semaphore_signal(
                barrier_sem, inc=1,
                device_id=(peer(b),), device_id_type=pl.DeviceIdType.MESH,
            )
        pl.semaphore_wait(barrier_sem, 3)

        def pchunk(c, lo, hi, out_dtype=jnp.bfloat16):
            a_c = a_ref[pl.ds(c * CHUNK, CHUNK), :].astype(jnp.bfloat16)
            p = lax.dot_general(
                a_c, b_bf[:, lo:hi], (((1,), (0,)), ((), ())),
                preferred_element_type=jnp.float32,
            )
            return p.astype(out_dtype) if out_dtype != jnp.float32 else p

        def send(src_at, dst_at, ssem_at, rsem_at, b):
            return pltpu.make_async_remote_copy(
                src_ref=src_at, dst_ref=dst_at,
                send_sem=ssem_at, recv_sem=rsem_at,
                device_id=(peer(b),), device_id_type=pl.DeviceIdType.MESH,
            )

        ap = peer(7)
        p0 = {}
        for b in LINKS:
            c0, c1 = COLS[b]
            b_bf[:, c0:c1] = b_ref[:, c0:c1].astype(jnp.bfloat16)
            ap_send[:, c0:c1] = pchunk(ap, c0, c1)
            p0[b] = send(ap_send.at[:, pl.ds(c0, c1 - c0)], p0_recv[b],
                         p0_ssem.at[li[b]], p0_rsem.at[li[b]], b)
            p0[b].start()

        def piece_rdma(sbuf, rbuf, ssem, rsem, l, pc):
            c0, c1 = PIECES[l][pc]
            return send(sbuf.at[li[l], :, pl.ds(c0, c1 - c0)],
                        rbuf.at[li[l], :, pl.ds(c0, c1 - c0)],
                        ssem.at[li[l], pc], rsem.at[li[l], pc], l)

        for l in LINKS:
            p1_send[li[l]] = pchunk(peer(FLIPINV[l]), 0, N)
            piece_rdma(p1_send, p1_recv, p1_ssem, p1_rsem, l, 0).start()
        for l in LINKS:
            a = 7 - FLIPINV[l]
            c0, c1 = COLS[a]
            p0[a].wait_recv()
            p1_send[li[l], :, c0:c1] = (
                p1_send[li[l], :, c0:c1] + p0_recv[a][...])
            piece_rdma(p1_send, p1_recv, p1_ssem, p1_rsem, l, 1).start()

        for lin, b in RELAY:
            p2_send[li[b]] = pchunk(peer(b), 0, N)
        for pc in range(2):
            for lin, b in RELAY:
                c0, c1 = PIECES[lin][pc]
                piece_rdma(p1_send, p1_recv, p1_ssem, p1_rsem, lin,
                           pc).wait_recv()
                p2_send[li[b], :, c0:c1] = (
                    p2_send[li[b], :, c0:c1] + p1_recv[li[lin], :, c0:c1])
                c = send(p2_send.at[li[b], :, pl.ds(c0, c1 - c0)],
                         p2_recv.at[li[b], :, pl.ds(c0, c1 - c0)],
                         p2_ssem.at[li[b], pc], p2_rsem.at[li[b], pc], b)
                c.start()

        acc = pchunk(my, 0, N, jnp.float32)
        for lin, b in RELAY:
            for pc in range(2):
                c0, c1 = PIECES[lin][pc]
                send(p2_send.at[li[b], :, pl.ds(c0, c1 - c0)],
                     p2_recv.at[li[b], :, pl.ds(c0, c1 - c0)],
                     p2_ssem.at[li[b], pc], p2_rsem.at[li[b], pc],
                     b).wait_recv()
            acc = acc + p2_recv[li[b]].astype(jnp.float32)
        out_ref[...] = acc

        for b in LINKS:
            p0[b].wait_send()
            for pc in range(2):
                piece_rdma(p1_send, p1_recv, p1_ssem, p1_rsem, b,
                           pc).wait_send()
        for lin, b in RELAY:
            for pc in range(2):
                c0, c1 = PIECES[lin][pc]
                send(p2_send.at[li[b], :, pl.ds(c0, c1 - c0)],
                     p2_recv.at[li[b], :, pl.ds(c0, c1 - c0)],
                     p2_ssem.at[li[b], pc], p2_rsem.at[li[b], pc],
                     b).wait_send()

    return pl.pallas_call(
        body,
        out_shape=jax.ShapeDtypeStruct((CHUNK, N), jnp.float32),
        in_specs=[
            pl.BlockSpec(memory_space=pltpu.VMEM),
            pl.BlockSpec(memory_space=pltpu.VMEM),
        ],
        out_specs=pl.BlockSpec(memory_space=pltpu.VMEM),
        scratch_shapes=[
            pltpu.VMEM((K, N), jnp.bfloat16),
            pltpu.VMEM((CHUNK, N), jnp.bfloat16),
            pltpu.VMEM((CHUNK, 768), jnp.bfloat16),
            pltpu.VMEM((CHUNK, 768), jnp.bfloat16),
            pltpu.VMEM((CHUNK, 512), jnp.bfloat16),
            pltpu.VMEM((3, CHUNK, N), jnp.bfloat16),
            pltpu.VMEM((3, CHUNK, N), jnp.bfloat16),
            pltpu.VMEM((3, CHUNK, N), jnp.bfloat16),
            pltpu.VMEM((3, CHUNK, N), jnp.bfloat16),
            pltpu.SemaphoreType.DMA((3,)),
            pltpu.SemaphoreType.DMA((3,)),
            pltpu.SemaphoreType.DMA((3, 2)),
            pltpu.SemaphoreType.DMA((3, 2)),
            pltpu.SemaphoreType.DMA((3, 2)),
            pltpu.SemaphoreType.DMA((3, 2)),
        ],
        compiler_params=pltpu.CompilerParams(collective_id=0),
    )(A, B)


# device time: 43157 ns/iter; 1.5513x vs baseline; 1.0311x over previous
import jax
import jax.numpy as jnp
from jax import lax
from jax.experimental import pallas as pl
from jax.experimental.pallas import tpu as pltpu

N_DEV = 8
LINKS = (4, 2, 1)
FLIPINV = {2: 3, 1: 5, 4: 6}
RELAY = ((2, 1), (1, 4), (4, 2))
COLS = {4: (0, 768), 2: (768, 1536), 1: (1536, 2048)}
PIECES = {
    4: ((0, 1536), (1536, 2048)),
    2: ((768, 2048), (0, 768)),
    1: ((0, 768), (768, 2048)),
}


def kernel(A, B):
    M, K = A.shape
    K2, N = B.shape
    assert K == K2
    CHUNK = M // N_DEV

    def body(a_ref, b_ref, out_ref, b_bf, ap_send,
             p0_recv4, p0_recv2, p0_recv1,
             p1_send, p1_recv, p2_send, p2_recv,
             p0_ssem, p0_rsem, p1_ssem, p1_rsem, p2_ssem, p2_rsem):
        my = lax.axis_index("i")
        p0_recv = {4: p0_recv4, 2: p0_recv2, 1: p0_recv1}
        li = {4: 0, 2: 1, 1: 2}

        z0 = lax.div(my, 4)
        q = lax.rem(my, 4)
        y0 = lax.div(q, 2)
        x0 = lax.rem(q + y0, 2)

        def peer(m):
            px = 1 - x0 if m & 4 else x0
            py = 1 - y0 if m & 2 else y0
            pz = 1 - z0 if m & 1 else z0
            return 4 * pz + 2 * py + lax.rem(px + py, 2)

        barrier_sem = pltpu.get_barrier_semaphore()
        for b in LINKS:
            pl.semaphore_signal(
                barrier_sem, inc=1,
                device_id=(peer(b),), device_id_type=pl.DeviceIdType.MESH,
            )
        pl.semaphore_wait(barrier_sem, 3)

        def pchunk(c, lo, hi, out_dtype=jnp.bfloat16):
            a_c = a_ref[pl.ds(c * CHUNK, CHUNK), :].astype(jnp.bfloat16)
            p = lax.dot_general(
                a_c, b_bf[:, lo:hi], (((1,), (0,)), ((), ())),
                preferred_element_type=jnp.float32,
            )
            return p.astype(out_dtype) if out_dtype != jnp.float32 else p

        def send(src_at, dst_at, ssem_at, rsem_at, b):
            return pltpu.make_async_remote_copy(
                src_ref=src_at, dst_ref=dst_at,
                send_sem=ssem_at, recv_sem=rsem_at,
                device_id=(peer(b),), device_id_type=pl.DeviceIdType.MESH,
            )

        ap = peer(7)
        p0 = {}
        for b in LINKS:
            c0, c1 = COLS[b]
            b_bf[:, c0:c1] = b_ref[:, c0:c1].astype(jnp.bfloat16)
            ap_send[:, c0:c1] = pchunk(ap, c0, c1)
            p0[b] = send(ap_send.at[:, pl.ds(c0, c1 - c0)], p0_recv[b],
                         p0_ssem.at[li[b]], p0_rsem.at[li[b]], b)
            p0[b].start()

        def piece_rdma(sbuf, rbuf, ssem, rsem, l, pc):
            c0, c1 = PIECES[l][pc]
            return send(sbuf.at[li[l], :, pl.ds(c0, c1 - c0)],
                        rbuf.at[li[l], :, pl.ds(c0, c1 - c0)],
                        ssem.at[li[l], pc], rsem.at[li[l], pc], l)

        for l in LINKS:
            p1_send[li[l]] = pchunk(peer(FLIPINV[l]), 0, N)
            piece_rdma(p1_send, p1_recv, p1_ssem, p1_rsem, l, 0).start()
        for l in LINKS:
            a = 7 - FLIPINV[l]
            c0, c1 = COLS[a]
            p0[a].wait_recv()
            p1_send[li[l], :, c0:c1] = (
                p1_send[li[l], :, c0:c1] + p0_recv[a][...])
            piece_rdma(p1_send, p1_recv, p1_ssem, p1_rsem, l, 1).start()

        for lin, b in RELAY:
            p2_send[li[b]] = pchunk(peer(b), 0, N)
        for pc in range(2):
            for lin, b in RELAY:
                c0, c1 = PIECES[lin][pc]
                piece_rdma(p1_send, p1_recv, p1_ssem, p1_rsem, lin,
                           pc).wait_recv()
                p2_send[li[b], :, c0:c1] = (
                    p2_send[li[b], :, c0:c1] + p1_recv[li[lin], :, c0:c1])
                c = send(p2_send.at[li[b], :, pl.ds(c0, c1 - c0)],
                         p2_recv.at[li[b], :, pl.ds(c0, c1 - c0)],
                         p2_ssem.at[li[b], pc], p2_rsem.at[li[b], pc], b)
                c.start()

        own = pchunk(my, 0, N, jnp.float32)

        def wait_p2(lin, b, pc):
            c0, c1 = PIECES[lin][pc]
            send(p2_send.at[li[b], :, pl.ds(c0, c1 - c0)],
                 p2_recv.at[li[b], :, pl.ds(c0, c1 - c0)],
                 p2_ssem.at[li[b], pc], p2_rsem.at[li[b], pc],
                 b).wait_recv()

        for (c0, c1), needs in (
            ((0, 768), ((2, 1, 1), (1, 4, 0), (4, 2, 0))),
            ((768, 1536), ((2, 1, 0), (1, 4, 1))),
            ((1536, 2048), ((4, 2, 1),)),
        ):
            for lin, b, pc in needs:
                wait_p2(lin, b, pc)
            out_ref[:, c0:c1] = (
                own[:, c0:c1]
                + p2_recv[li[1], :, c0:c1].astype(jnp.float32)
                + p2_recv[li[4], :, c0:c1].astype(jnp.float32)
                + p2_recv[li[2], :, c0:c1].astype(jnp.float32))

        for b in LINKS:
            p0[b].wait_send()
            for pc in range(2):
                piece_rdma(p1_send, p1_recv, p1_ssem, p1_rsem, b,
                           pc).wait_send()
        for lin, b in RELAY:
            for pc in range(2):
                c0, c1 = PIECES[lin][pc]
                send(p2_send.at[li[b], :, pl.ds(c0, c1 - c0)],
                     p2_recv.at[li[b], :, pl.ds(c0, c1 - c0)],
                     p2_ssem.at[li[b], pc], p2_rsem.at[li[b], pc],
                     b).wait_send()

    return pl.pallas_call(
        body,
        out_shape=jax.ShapeDtypeStruct((CHUNK, N), jnp.float32),
        in_specs=[
            pl.BlockSpec(memory_space=pltpu.VMEM),
            pl.BlockSpec(memory_space=pltpu.VMEM),
        ],
        out_specs=pl.BlockSpec(memory_space=pltpu.VMEM),
        scratch_shapes=[
            pltpu.VMEM((K, N), jnp.bfloat16),
            pltpu.VMEM((CHUNK, N), jnp.bfloat16),
            pltpu.VMEM((CHUNK, 768), jnp.bfloat16),
            pltpu.VMEM((CHUNK, 768), jnp.bfloat16),
            pltpu.VMEM((CHUNK, 512), jnp.bfloat16),
            pltpu.VMEM((3, CHUNK, N), jnp.bfloat16),
            pltpu.VMEM((3, CHUNK, N), jnp.bfloat16),
            pltpu.VMEM((3, CHUNK, N), jnp.bfloat16),
            pltpu.VMEM((3, CHUNK, N), jnp.bfloat16),
            pltpu.SemaphoreType.DMA((3,)),
            pltpu.SemaphoreType.DMA((3,)),
            pltpu.SemaphoreType.DMA((3, 2)),
            pltpu.SemaphoreType.DMA((3, 2)),
            pltpu.SemaphoreType.DMA((3, 2)),
            pltpu.SemaphoreType.DMA((3, 2)),
        ],
        compiler_params=pltpu.CompilerParams(collective_id=0),
    )(A, B)


# device time: 42263 ns/iter; 1.5842x vs baseline; 1.0212x over previous
import jax
import jax.numpy as jnp
from jax import lax
from jax.experimental import pallas as pl
from jax.experimental.pallas import tpu as pltpu

N_DEV = 8
LINKS = (4, 2, 1)
FLIPINV = {2: 3, 1: 5, 4: 6}
RELAY = ((2, 1), (1, 4), (4, 2))
COLS = {4: (0, 768), 2: (768, 1536), 1: (1536, 2048)}
PIECES = {
    4: ((0, 1536), (1536, 2048)),
    2: ((768, 2048), (0, 768)),
    1: ((0, 768), (768, 2048)),
}


def kernel(A, B):
    M, K = A.shape
    K2, N = B.shape
    assert K == K2
    CHUNK = M // N_DEV

    def body(a_ref, b_ref, out_ref, b_bf, ap_send,
             p0_recv4, p0_recv2, p0_recv1,
             p1_send, p1_recv, p2_send, p2_recv,
             p0_ssem, p0_rsem, p1_ssem, p1_rsem, p2_ssem, p2_rsem):
        my = lax.axis_index("i")
        p0_recv = {4: p0_recv4, 2: p0_recv2, 1: p0_recv1}
        li = {4: 0, 2: 1, 1: 2}

        z0 = lax.div(my, 4)
        q = lax.rem(my, 4)
        y0 = lax.div(q, 2)
        x0 = lax.rem(q + y0, 2)

        def peer(m):
            px = 1 - x0 if m & 4 else x0
            py = 1 - y0 if m & 2 else y0
            pz = 1 - z0 if m & 1 else z0
            return 4 * pz + 2 * py + lax.rem(px + py, 2)

        barrier_sem = pltpu.get_barrier_semaphore()
        for b in LINKS:
            pl.semaphore_signal(
                barrier_sem, inc=1,
                device_id=(peer(b),), device_id_type=pl.DeviceIdType.MESH,
            )
        pl.semaphore_wait(barrier_sem, 3)

        def pchunk(c, lo, hi, out_dtype=jnp.bfloat16):
            a_c = a_ref[pl.ds(c * CHUNK, CHUNK), :].astype(jnp.bfloat16)
            p = lax.dot_general(
                a_c, b_bf[:, lo:hi], (((1,), (0,)), ((), ())),
                preferred_element_type=jnp.float32,
            )
            return p.astype(out_dtype) if out_dtype != jnp.float32 else p

        def send(src_at, dst_at, ssem_at, rsem_at, b):
            return pltpu.make_async_remote_copy(
                src_ref=src_at, dst_ref=dst_at,
                send_sem=ssem_at, recv_sem=rsem_at,
                device_id=(peer(b),), device_id_type=pl.DeviceIdType.MESH,
            )

        ap = peer(7)
        p0 = {}
        for b in LINKS:
            c0, c1 = COLS[b]
            b_bf[:, c0:c1] = b_ref[:, c0:c1].astype(jnp.bfloat16)
            ap_send[:, c0:c1] = pchunk(ap, c0, c1)
            p0[b] = send(ap_send.at[:, pl.ds(c0, c1 - c0)], p0_recv[b],
                         p0_ssem.at[li[b]], p0_rsem.at[li[b]], b)
            p0[b].start()

        def piece_rdma(sbuf, rbuf, ssem, rsem, l, pc):
            c0, c1 = PIECES[l][pc]
            return send(sbuf.at[li[l], :, pl.ds(c0, c1 - c0)],
                        rbuf.at[li[l], :, pl.ds(c0, c1 - c0)],
                        ssem.at[li[l], pc], rsem.at[li[l], pc], l)

        for l in LINKS:
            c0, c1 = PIECES[l][0]
            p1_send[li[l], :, c0:c1] = pchunk(peer(FLIPINV[l]), c0, c1)
            piece_rdma(p1_send, p1_recv, p1_ssem, p1_rsem, l, 0).start()
        for l in LINKS:
            d0, d1 = PIECES[l][1]
            p1_send[li[l], :, d0:d1] = pchunk(peer(FLIPINV[l]), d0, d1)
            a = 7 - FLIPINV[l]
            c0, c1 = COLS[a]
            p0[a].wait_recv()
            p1_send[li[l], :, c0:c1] = (
                p1_send[li[l], :, c0:c1] + p0_recv[a][...])
            piece_rdma(p1_send, p1_recv, p1_ssem, p1_rsem, l, 1).start()

        for lin, b in RELAY:
            p2_send[li[b]] = pchunk(peer(b), 0, N)
        for pc in range(2):
            for lin, b in RELAY:
                c0, c1 = PIECES[lin][pc]
                piece_rdma(p1_send, p1_recv, p1_ssem, p1_rsem, lin,
                           pc).wait_recv()
                p2_send[li[b], :, c0:c1] = (
                    p2_send[li[b], :, c0:c1] + p1_recv[li[lin], :, c0:c1])
                c = send(p2_send.at[li[b], :, pl.ds(c0, c1 - c0)],
                         p2_recv.at[li[b], :, pl.ds(c0, c1 - c0)],
                         p2_ssem.at[li[b], pc], p2_rsem.at[li[b], pc], b)
                c.start()

        own = pchunk(my, 0, N, jnp.float32)

        def wait_p2(lin, b, pc):
            c0, c1 = PIECES[lin][pc]
            send(p2_send.at[li[b], :, pl.ds(c0, c1 - c0)],
                 p2_recv.at[li[b], :, pl.ds(c0, c1 - c0)],
                 p2_ssem.at[li[b], pc], p2_rsem.at[li[b], pc],
                 b).wait_recv()

        for (c0, c1), needs in (
            ((0, 768), ((2, 1, 1), (1, 4, 0), (4, 2, 0))),
            ((768, 1536), ((2, 1, 0), (1, 4, 1))),
            ((1536, 2048), ((4, 2, 1),)),
        ):
            for lin, b, pc in needs:
                wait_p2(lin, b, pc)
            out_ref[:, c0:c1] = (
                own[:, c0:c1]
                + p2_recv[li[1], :, c0:c1].astype(jnp.float32)
                + p2_recv[li[4], :, c0:c1].astype(jnp.float32)
                + p2_recv[li[2], :, c0:c1].astype(jnp.float32))

        for b in LINKS:
            p0[b].wait_send()
            for pc in range(2):
                piece_rdma(p1_send, p1_recv, p1_ssem, p1_rsem, b,
                           pc).wait_send()
        for lin, b in RELAY:
            for pc in range(2):
                c0, c1 = PIECES[lin][pc]
                send(p2_send.at[li[b], :, pl.ds(c0, c1 - c0)],
                     p2_recv.at[li[b], :, pl.ds(c0, c1 - c0)],
                     p2_ssem.at[li[b], pc], p2_rsem.at[li[b], pc],
                     b).wait_send()

    return pl.pallas_call(
        body,
        out_shape=jax.ShapeDtypeStruct((CHUNK, N), jnp.float32),
        in_specs=[
            pl.BlockSpec(memory_space=pltpu.VMEM),
            pl.BlockSpec(memory_space=pltpu.VMEM),
        ],
        out_specs=pl.BlockSpec(memory_space=pltpu.VMEM),
        scratch_shapes=[
            pltpu.VMEM((K, N), jnp.bfloat16),
            pltpu.VMEM((CHUNK, N), jnp.bfloat16),
            pltpu.VMEM((CHUNK, 768), jnp.bfloat16),
            pltpu.VMEM((CHUNK, 768), jnp.bfloat16),
            pltpu.VMEM((CHUNK, 512), jnp.bfloat16),
            pltpu.VMEM((3, CHUNK, N), jnp.bfloat16),
            pltpu.VMEM((3, CHUNK, N), jnp.bfloat16),
            pltpu.VMEM((3, CHUNK, N), jnp.bfloat16),
            pltpu.VMEM((3, CHUNK, N), jnp.bfloat16),
            pltpu.SemaphoreType.DMA((3,)),
            pltpu.SemaphoreType.DMA((3,)),
            pltpu.SemaphoreType.DMA((3, 2)),
            pltpu.SemaphoreType.DMA((3, 2)),
            pltpu.SemaphoreType.DMA((3, 2)),
            pltpu.SemaphoreType.DMA((3, 2)),
        ],
        compiler_params=pltpu.CompilerParams(collective_id=0),
    )(A, B)


# device time: 41455 ns/iter; 1.6150x vs baseline; 1.0195x over previous
import jax
import jax.numpy as jnp
from jax import lax
from jax.experimental import pallas as pl
from jax.experimental.pallas import tpu as pltpu

N_DEV = 8
LINKS = (4, 2, 1)
FLIPINV = {2: 3, 1: 5, 4: 6}
RELAY = ((2, 1), (1, 4), (4, 2))
COLS = {4: (0, 768), 2: (768, 1536), 1: (1536, 2048)}
PIECES = {
    4: ((0, 1536), (1536, 2048)),
    2: ((768, 2048), (0, 768)),
    1: ((0, 768), (768, 2048)),
}


def kernel(A, B):
    M, K = A.shape
    K2, N = B.shape
    assert K == K2
    CHUNK = M // N_DEV

    def body(a_ref, b_ref, out_ref, b_bf, ap_send,
             p0_recv4, p0_recv2, p0_recv1,
             p1_send, p1_recv, p2_send, p2_recv,
             p0_ssem, p0_rsem, p1_ssem, p1_rsem, p2_ssem, p2_rsem):
        my = lax.axis_index("i")
        p0_recv = {4: p0_recv4, 2: p0_recv2, 1: p0_recv1}
        li = {4: 0, 2: 1, 1: 2}

        z0 = lax.div(my, 4)
        q = lax.rem(my, 4)
        y0 = lax.div(q, 2)
        x0 = lax.rem(q + y0, 2)

        def peer(m):
            px = 1 - x0 if m & 4 else x0
            py = 1 - y0 if m & 2 else y0
            pz = 1 - z0 if m & 1 else z0
            return 4 * pz + 2 * py + lax.rem(px + py, 2)

        barrier_sem = pltpu.get_barrier_semaphore()
        for b in LINKS:
            pl.semaphore_signal(
                barrier_sem, inc=1,
                device_id=(peer(b),), device_id_type=pl.DeviceIdType.MESH,
            )

        def pchunk(c, lo, hi, out_dtype=jnp.bfloat16):
            a_c = a_ref[pl.ds(c * CHUNK, CHUNK), :].astype(jnp.bfloat16)
            p = lax.dot_general(
                a_c, b_bf[:, lo:hi], (((1,), (0,)), ((), ())),
                preferred_element_type=jnp.float32,
            )
            return p.astype(out_dtype) if out_dtype != jnp.float32 else p

        def send(src_at, dst_at, ssem_at, rsem_at, b):
            return pltpu.make_async_remote_copy(
                src_ref=src_at, dst_ref=dst_at,
                send_sem=ssem_at, recv_sem=rsem_at,
                device_id=(peer(b),), device_id_type=pl.DeviceIdType.MESH,
            )

        ap = peer(7)
        p0 = {}
        for i, b in enumerate(LINKS):
            c0, c1 = COLS[b]
            b_bf[:, c0:c1] = b_ref[:, c0:c1].astype(jnp.bfloat16)
            ap_send[:, c0:c1] = pchunk(ap, c0, c1)
            if i == 0:
                pl.semaphore_wait(barrier_sem, 3)
            p0[b] = send(ap_send.at[:, pl.ds(c0, c1 - c0)], p0_recv[b],
                         p0_ssem.at[li[b]], p0_rsem.at[li[b]], b)
            p0[b].start()

        def piece_rdma(sbuf, rbuf, ssem, rsem, l, pc):
            c0, c1 = PIECES[l][pc]
            return send(sbuf.at[li[l], :, pl.ds(c0, c1 - c0)],
                        rbuf.at[li[l], :, pl.ds(c0, c1 - c0)],
                        ssem.at[li[l], pc], rsem.at[li[l], pc], l)

        for l in LINKS:
            c0, c1 = PIECES[l][0]
            p1_send[li[l], :, c0:c1] = pchunk(peer(FLIPINV[l]), c0, c1)
            piece_rdma(p1_send, p1_recv, p1_ssem, p1_rsem, l, 0).start()
        for l in LINKS:
            d0, d1 = PIECES[l][1]
            p1_send[li[l], :, d0:d1] = pchunk(peer(FLIPINV[l]), d0, d1)
            a = 7 - FLIPINV[l]
            c0, c1 = COLS[a]
            p0[a].wait_recv()
            p1_send[li[l], :, c0:c1] = (
                p1_send[li[l], :, c0:c1] + p0_recv[a][...])
            piece_rdma(p1_send, p1_recv, p1_ssem, p1_rsem, l, 1).start()

        for lin, b in RELAY:
            p2_send[li[b]] = pchunk(peer(b), 0, N)
        for pc in range(2):
            for lin, b in RELAY:
                c0, c1 = PIECES[lin][pc]
                piece_rdma(p1_send, p1_recv, p1_ssem, p1_rsem, lin,
                           pc).wait_recv()
                p2_send[li[b], :, c0:c1] = (
                    p2_send[li[b], :, c0:c1] + p1_recv[li[lin], :, c0:c1])
                c = send(p2_send.at[li[b], :, pl.ds(c0, c1 - c0)],
                         p2_recv.at[li[b], :, pl.ds(c0, c1 - c0)],
                         p2_ssem.at[li[b], pc], p2_rsem.at[li[b], pc], b)
                c.start()

        own = pchunk(my, 0, N, jnp.float32)

        def wait_p2(lin, b, pc):
            c0, c1 = PIECES[lin][pc]
            send(p2_send.at[li[b], :, pl.ds(c0, c1 - c0)],
                 p2_recv.at[li[b], :, pl.ds(c0, c1 - c0)],
                 p2_ssem.at[li[b], pc], p2_rsem.at[li[b], pc],
                 b).wait_recv()

        for (c0, c1), needs in (
            ((0, 768), ((2, 1, 1), (1, 4, 0), (4, 2, 0))),
            ((768, 1536), ((2, 1, 0), (1, 4, 1))),
            ((1536, 2048), ((4, 2, 1),)),
        ):
            for lin, b, pc in needs:
                wait_p2(lin, b, pc)
            out_ref[:, c0:c1] = (
                own[:, c0:c1]
                + p2_recv[li[1], :, c0:c1].astype(jnp.float32)
                + p2_recv[li[4], :, c0:c1].astype(jnp.float32)
                + p2_recv[li[2], :, c0:c1].astype(jnp.float32))

        for b in LINKS:
            p0[b].wait_send()
            for pc in range(2):
                piece_rdma(p1_send, p1_recv, p1_ssem, p1_rsem, b,
                           pc).wait_send()
        for lin, b in RELAY:
            for pc in range(2):
                c0, c1 = PIECES[lin][pc]
                send(p2_send.at[li[b], :, pl.ds(c0, c1 - c0)],
                     p2_recv.at[li[b], :, pl.ds(c0, c1 - c0)],
                     p2_ssem.at[li[b], pc], p2_rsem.at[li[b], pc],
                     b).wait_send()

    return pl.pallas_call(
        body,
        out_shape=jax.ShapeDtypeStruct((CHUNK, N), jnp.float32),
        in_specs=[
            pl.BlockSpec(memory_space=pltpu.VMEM),
            pl.BlockSpec(memory_space=pltpu.VMEM),
        ],
        out_specs=pl.BlockSpec(memory_space=pltpu.VMEM),
        scratch_shapes=[
            pltpu.VMEM((K, N), jnp.bfloat16),
            pltpu.VMEM((CHUNK, N), jnp.bfloat16),
            pltpu.VMEM((CHUNK, 768), jnp.bfloat16),
            pltpu.VMEM((CHUNK, 768), jnp.bfloat16),
            pltpu.VMEM((CHUNK, 512), jnp.bfloat16),
            pltpu.VMEM((3, CHUNK, N), jnp.bfloat16),
            pltpu.VMEM((3, CHUNK, N), jnp.bfloat16),
            pltpu.VMEM((3, CHUNK, N), jnp.bfloat16),
            pltpu.VMEM((3, CHUNK, N), jnp.bfloat16),
            pltpu.SemaphoreType.DMA((3,)),
            pltpu.SemaphoreType.DMA((3,)),
            pltpu.SemaphoreType.DMA((3, 2)),
            pltpu.SemaphoreType.DMA((3, 2)),
            pltpu.SemaphoreType.DMA((3, 2)),
            pltpu.SemaphoreType.DMA((3, 2)),
        ],
        compiler_params=pltpu.CompilerParams(collective_id=0),
    )(A, B)
